# baseline (device time: 17435 ns/iter reference)
import jax
import jax.numpy as jnp
from jax import lax
from jax.experimental import pallas as pl
from jax.experimental.pallas import tpu as pltpu

N_DEV = 4
B, SQ, SKV = 2, 256, 256
HQ, DH = 16, 64
H_LOC = HQ // N_DEV
D_MODEL = 512
HALF = D_MODEL // 2
WINDOW = 128
BF = jnp.bfloat16


def kernel(x, Wq, K_ext, V_ext, Wo):
    my = lax.axis_index("i")
    k_mine = jnp.moveaxis(
        lax.dynamic_slice_in_dim(K_ext, my * H_LOC, H_LOC, axis=2), 2, 0
    ).astype(BF)
    v_mine = jnp.moveaxis(
        lax.dynamic_slice_in_dim(V_ext, my * H_LOC, H_LOC, axis=2), 2, 0
    ).astype(BF)

    def body(x_ref, wq_ref, k_ref, v_ref, wo_ref, out_ref,
             accA_ref, accB_ref, cA_ref, cB_ref, sA, rA, sB, rB):
        my_pos = lax.axis_index("i")
        x_partner = 3 - my_pos
        y_partner = my_pos ^ 1

        barrier_sem = pltpu.get_barrier_semaphore()
        for nbr in (x_partner, y_partner):
            pl.semaphore_signal(barrier_sem, inc=1, device_id=(nbr,),
                                device_id_type=pl.DeviceIdType.MESH)
        pl.semaphore_wait(barrier_sem, 2)

        qi = lax.broadcasted_iota(jnp.int32, (SQ, SKV), 0)
        ki = lax.broadcasted_iota(jnp.int32, (SQ, SKV), 1)
        mask = jnp.abs(qi - ki) <= WINDOW

        def mk(c_ref, sems, rems, stage, b, partner):
            return pltpu.make_async_remote_copy(
                src_ref=c_ref.at[2 * stage, b],
                dst_ref=c_ref.at[2 * stage + 1, b],
                send_sem=sems.at[stage, b],
                recv_sem=rems.at[stage, b],
                device_id=(partner,), device_id_type=pl.DeviceIdType.MESH)

        q_all = jnp.dot(x_ref[...].reshape(B * SQ, D_MODEL), wq_ref[...],
                        preferred_element_type=jnp.float32)

        stage1 = {}
        for b in range(B):
            q = q_all[b * SQ:(b + 1) * SQ]
            ctx_parts = []
            for h in range(H_LOC):
                qh = q[:, h * DH:(h + 1) * DH].astype(BF)
                kh = k_ref[h, b]
                vh = v_ref[h, b]
                s = lax.dot_general(
                    qh, kh, (((1,), (1,)), ((), ())),
                    preferred_element_type=jnp.float32) * 0.125
                w = jnp.where(mask, jnp.exp(s), 0.0)
                recip = 1.0 / jnp.sum(w, axis=1, keepdims=True)
                ctx_parts.append(
                    jnp.dot(w.astype(BF), vh,
                            preferred_element_type=jnp.float32) * recip)
            ctx = jnp.concatenate(ctx_parts, axis=1).astype(BF)
            pA = jnp.dot(ctx, wo_ref[:, :HALF],
                         preferred_element_type=jnp.float32)
            accA_ref[b] = pA
            cA_ref[0, b] = pA.astype(BF)
            a1 = mk(cA_ref, sA, rA, 0, b, x_partner)
            a1.start()
            pB = jnp.dot(ctx, wo_ref[:, HALF:],
                         preferred_element_type=jnp.float32)
            accB_ref[b] = pB
            cB_ref[0, b] = pB.astype(BF)
            b1 = mk(cB_ref, sB, rB, 0, b, y_partner)
            b1.start()
            stage1[b] = (a1, b1)

        stage2 = {}
        for b in range(B):
            a1, b1 = stage1[b]
            a1.wait()
            accA_ref[b] = accA_ref[b] + cA_ref[1, b].astype(jnp.float32)
            cA_ref[2, b] = accA_ref[b].astype(BF)
            a2 = mk(cA_ref, sA, rA, 1, b, y_partner)
            a2.start()
            b1.wait()
            accB_ref[b] = accB_ref[b] + cB_ref[1, b].astype(jnp.float32)
            cB_ref[2, b] = accB_ref[b].astype(BF)
            b2 = mk(cB_ref, sB, rB, 1, b, x_partner)
            b2.start()
            stage2[b] = (a2, b2)

        for b in range(B):
            a2, b2 = stage2[b]
            a2.wait()
            out_ref[b, :, :HALF] = accA_ref[b] + cA_ref[3, b].astype(jnp.float32)
            b2.wait()
            out_ref[b, :, HALF:] = accB_ref[b] + cB_ref[3, b].astype(jnp.float32)

    return pl.pallas_call(
        body,
        out_shape=jax.ShapeDtypeStruct((B, SQ, D_MODEL), jnp.float32),
        in_specs=[pl.BlockSpec(memory_space=pltpu.VMEM)] * 5,
        out_specs=pl.BlockSpec(memory_space=pltpu.VMEM),
        scratch_shapes=[
            pltpu.VMEM((B, SQ, HALF), jnp.float32),
            pltpu.VMEM((B, SQ, HALF), jnp.float32),
            pltpu.VMEM((4, B, SQ, HALF), BF),
            pltpu.VMEM((4, B, SQ, HALF), BF),
            pltpu.SemaphoreType.DMA((2, B)),
            pltpu.SemaphoreType.DMA((2, B)),
            pltpu.SemaphoreType.DMA((2, B)),
            pltpu.SemaphoreType.DMA((2, B)),
        ],
        compiler_params=pltpu.CompilerParams(collective_id=0),
    )(x.astype(BF), Wq.astype(BF), k_mine, v_mine, Wo.astype(BF))
